# baseline (speedup 1.0000x reference)
import functools

import jax
import jax.numpy as jnp
from jax import lax
from jax.experimental import pallas as pl
from jax.experimental.pallas import tpu as pltpu
from jax.experimental.pallas import tpu_sc as plsc

MODEL_DIM = 128
STEP_DIM = MODEL_DIM // 2
SIGNAL_DIM = MODEL_DIM - STEP_DIM
BATCH = 16384
STEP_VOCAB = 20

_NC, _NS = 2, 16
_NW = _NC * _NS
_BPW = BATCH // _NW

_LSPAN = 524232
_CCHUNK = 4
_NROUND = SIGNAL_DIM // _CCHUNK


def _emb_kernel(step_idx_hbm, sig_idx_hbm, step_tab_hbm, sig_lines_hbm,
                out_hbm, ridx_v, sidx_v, posl_v, lane_v, stab_v, lbuf_v,
                sobuf_v, robuf_v, sema, semb, osema, osemb, ssem):
    wid = lax.axis_index("s") * _NC + lax.axis_index("c")
    base = wid * _BPW
    u0 = base // 128
    iota16 = lax.iota(jnp.int32, 16)
    gsems = (sema, semb)
    osems = (osema, osemb)

    pltpu.sync_copy(sig_idx_hbm.at[pl.ds(base, _BPW)], ridx_v)
    pltpu.sync_copy(step_idx_hbm.at[pl.ds(base, _BPW)], sidx_v)
    pltpu.sync_copy(step_tab_hbm, stab_v)

    def _pos_body(k, carry):
        r = ridx_v[pl.ds(k * 16, 16)]
        posl_v[pl.ds(k * 16, 16)] = ((r >> 7) << 6) | ((r >> 4) & 7)
        lane_v[pl.ds(k * 16, 16)] = r & 15
        return carry

    lax.fori_loop(0, _BPW // 16, _pos_body, 0)

    def _fire(n):
        buf = n % 2
        cps = []
        for dc in range(_CCHUNK):
            c = n * _CCHUNK + dc
            src = sig_lines_hbm.at[
                pl.ds((c // 8) * 524288 + (c % 8) * 8, _LSPAN)]
            cps.append(pltpu.async_copy(
                src.at[posl_v],
                lbuf_v.at[buf, pl.ds(dc * _BPW, _BPW)], gsems[buf]))
        return cps

    pend = {0: _fire(0), 1: _fire(1)}

    def _step_body(k, carry):
        s = sidx_v[pl.ds(k * 16, 16)]
        ub = k >> 3
        off = (k & 7) << 5
        for cp in range(STEP_DIM // 2):
            g0 = plsc.load_gather(
                stab_v, [s, jnp.full((16,), 2 * cp, jnp.int32)])
            g1 = plsc.load_gather(
                stab_v, [s, jnp.full((16,), 2 * cp + 1, jnp.int32)])
            pk = plsc.pack(g0, g1, format=plsc.PackFormat.INTERLEAVED)
            sobuf_v[cp // 4, ub, pl.ds((cp % 4) * 256 + off, 32)] = pk
        return carry

    lax.fori_loop(0, _BPW // 16, _step_body, 0)
    step_flush = []
    for tt in range(8):
        step_flush.append(pltpu.async_copy(
            sobuf_v.at[tt], out_hbm.at[tt, pl.ds(u0, 4), :], ssem))

    oflush = {}
    for n in range(_NROUND):
        buf = n % 2
        for cp in pend.pop(n):
            cp.wait()
        if n - 2 in oflush:
            oflush.pop(n - 2).wait()

        def _ext_body(k, carry, buf=buf):
            lane = lane_v[pl.ds(k * 16, 16)]
            row0 = k * 16 + iota16
            ub = k >> 3
            off = (k & 7) << 5
            for p in range(_CCHUNK // 2):
                g0 = plsc.load_gather(
                    lbuf_v.at[buf], [row0 + (2 * p) * _BPW, lane])
                g1 = plsc.load_gather(
                    lbuf_v.at[buf], [row0 + (2 * p + 1) * _BPW, lane])
                pk = plsc.pack(g0, g1, format=plsc.PackFormat.INTERLEAVED)
                robuf_v[buf, ub, pl.ds(p * 256 + off, 32)] = pk
            return carry

        lax.fori_loop(0, _BPW // 16, _ext_body, 0)
        if n + 2 < _NROUND:
            pend[n + 2] = _fire(n + 2)
        oflush[n] = pltpu.async_copy(
            robuf_v.at[buf],
            out_hbm.at[8 + n // 2, pl.ds(u0, 4),
                       pl.ds((n % 2) * 512, 512)], osems[buf])

    for n in sorted(oflush):
        oflush[n].wait()
    for cp in step_flush:
        cp.wait()


@jax.jit
def _lookup(step_idx, sig_idx, step_tab, sig_lines):
    k = functools.partial(
        pl.kernel,
        out_type=jax.ShapeDtypeStruct((16, 128, 1024), jnp.bfloat16),
        mesh=plsc.VectorSubcoreMesh(core_axis_name="c", subcore_axis_name="s"),
        compiler_params=pltpu.CompilerParams(
            use_tc_tiling_on_sc=False, needs_layout_passes=False),
        scratch_types=[
            pltpu.VMEM((_BPW,), jnp.int32),
            pltpu.VMEM((_BPW,), jnp.int32),
            pltpu.VMEM((_BPW,), jnp.int32),
            pltpu.VMEM((_BPW,), jnp.int32),
            pltpu.VMEM((STEP_VOCAB, STEP_DIM), jnp.float32),
            pltpu.VMEM((2, _CCHUNK * _BPW, 16), jnp.float32),
            pltpu.VMEM((8, 4, 1024), jnp.bfloat16),
            pltpu.VMEM((2, 4, 512), jnp.bfloat16),
            pltpu.SemaphoreType.DMA,
            pltpu.SemaphoreType.DMA,
            pltpu.SemaphoreType.DMA,
            pltpu.SemaphoreType.DMA,
            pltpu.SemaphoreType.DMA,
        ],
    )(_emb_kernel)
    return k(step_idx, sig_idx, step_tab, sig_lines)


def kernel(step_levels, signal_levels, step_embedding, signal_embedding):
    step_idx = jnp.asarray(step_levels, dtype=jnp.int32)
    sig_idx = jnp.asarray(signal_levels, dtype=jnp.int32)
    sig_lines = (signal_embedding.T.reshape(8, 8, 8192, 128)
                 .transpose(0, 2, 1, 3).reshape(-1, 16))
    out3 = _lookup(step_idx, sig_idx, step_embedding, sig_lines)
    out5 = out3.reshape(16, 128, 4, 128, 2)
    return out5.transpose(1, 3, 0, 2, 4).reshape(BATCH, MODEL_DIM)


# speedup vs baseline: 1.1440x; 1.1440x over previous
import functools

import jax
import jax.numpy as jnp
from jax import lax
from jax.experimental import pallas as pl
from jax.experimental.pallas import tpu as pltpu
from jax.experimental.pallas import tpu_sc as plsc

MODEL_DIM = 128
STEP_DIM = MODEL_DIM // 2
SIGNAL_DIM = MODEL_DIM - STEP_DIM
BATCH = 16384
STEP_VOCAB = 20

_NC, _NS = 2, 16
_NW = _NC * _NS
_BPW = BATCH // _NW

_LSPAN = 524232
_CCHUNK = 4
_NROUND = SIGNAL_DIM // _CCHUNK


def _emb_kernel(step_idx_hbm, sig_idx_hbm, step_tab_hbm, sig_lines_hbm,
                out_hbm, ridx_v, sidx_v, posl_v, lane_v, stab_v, lbuf_v,
                sobuf_v, robuf_v, sema, semb, osema, osemb, ssem):
    wid = lax.axis_index("s") * _NC + lax.axis_index("c")
    base = wid * _BPW
    iota16 = lax.iota(jnp.int32, 16)
    gsems = (sema, semb)
    osems = (osema, osemb)

    pltpu.sync_copy(sig_idx_hbm.at[pl.ds(base, _BPW)], ridx_v)
    pltpu.sync_copy(step_idx_hbm.at[pl.ds(base, _BPW)], sidx_v)
    pltpu.sync_copy(step_tab_hbm, stab_v)

    def _pos_body(k, carry):
        r = ridx_v[pl.ds(k * 16, 16)]
        posl_v[pl.ds(k * 16, 16)] = ((r >> 7) << 6) | ((r >> 4) & 7)
        lane_v[pl.ds(k * 16, 16)] = r & 15
        return carry

    lax.fori_loop(0, _BPW // 16, _pos_body, 0)

    def _fire(n):
        buf = n % 2
        cps = []
        for dc in range(_CCHUNK):
            c = n * _CCHUNK + dc
            src = sig_lines_hbm.at[
                pl.ds((c // 8) * 524288 + (c % 8) * 8, _LSPAN)]
            cps.append(pltpu.async_copy(
                src.at[posl_v],
                lbuf_v.at[buf, pl.ds(dc * _BPW, _BPW)], gsems[buf]))
        return cps

    pend = {0: _fire(0), 1: _fire(1)}

    def _step_body(k, carry):
        se = plsc.load_gather(sidx_v, [k * 32 + 2 * iota16])
        so = plsc.load_gather(sidx_v, [k * 32 + 2 * iota16 + 1])
        for c in range(STEP_DIM):
            cc = jnp.full((16,), c, jnp.int32)
            g0 = plsc.load_gather(stab_v, [se, cc])
            g1 = plsc.load_gather(stab_v, [so, cc])
            pk = plsc.pack(g0, g1, format=plsc.PackFormat.INTERLEAVED)
            sobuf_v[c, pl.ds(k * 32, 32)] = pk
        return carry

    lax.fori_loop(0, _BPW // 32, _step_body, 0)
    step_flush = pltpu.async_copy(
        sobuf_v, out_hbm.at[pl.ds(0, STEP_DIM), pl.ds(base, _BPW)], ssem)

    oflush = {}
    for n in range(_NROUND):
        buf = n % 2
        for cp in pend.pop(n):
            cp.wait()
        if n - 2 in oflush:
            oflush.pop(n - 2).wait()

        def _ext_body(k, carry, buf=buf):
            le = plsc.load_gather(lane_v, [k * 32 + 2 * iota16])
            lo = plsc.load_gather(lane_v, [k * 32 + 2 * iota16 + 1])
            rowe = k * 32 + 2 * iota16
            rowo = rowe + 1
            for dc in range(_CCHUNK):
                g0 = plsc.load_gather(lbuf_v.at[buf], [rowe + dc * _BPW, le])
                g1 = plsc.load_gather(lbuf_v.at[buf], [rowo + dc * _BPW, lo])
                pk = plsc.pack(g0, g1, format=plsc.PackFormat.INTERLEAVED)
                robuf_v[buf, dc, pl.ds(k * 32, 32)] = pk
            return carry

        lax.fori_loop(0, _BPW // 32, _ext_body, 0)
        if n + 2 < _NROUND:
            pend[n + 2] = _fire(n + 2)
        oflush[n] = pltpu.async_copy(
            robuf_v.at[buf],
            out_hbm.at[pl.ds(STEP_DIM + n * _CCHUNK, _CCHUNK),
                       pl.ds(base, _BPW)], osems[buf])

    for n in sorted(oflush):
        oflush[n].wait()
    step_flush.wait()


@jax.jit
def _lookup(step_idx, sig_idx, step_tab, sig_lines):
    k = functools.partial(
        pl.kernel,
        out_type=jax.ShapeDtypeStruct((MODEL_DIM, BATCH), jnp.bfloat16),
        mesh=plsc.VectorSubcoreMesh(core_axis_name="c", subcore_axis_name="s"),
        compiler_params=pltpu.CompilerParams(
            use_tc_tiling_on_sc=False, needs_layout_passes=False),
        scratch_types=[
            pltpu.VMEM((_BPW,), jnp.int32),
            pltpu.VMEM((_BPW,), jnp.int32),
            pltpu.VMEM((_BPW,), jnp.int32),
            pltpu.VMEM((_BPW,), jnp.int32),
            pltpu.VMEM((STEP_VOCAB, STEP_DIM), jnp.float32),
            pltpu.VMEM((2, _CCHUNK * _BPW, 16), jnp.float32),
            pltpu.VMEM((STEP_DIM, _BPW), jnp.bfloat16),
            pltpu.VMEM((2, _CCHUNK, _BPW), jnp.bfloat16),
            pltpu.SemaphoreType.DMA,
            pltpu.SemaphoreType.DMA,
            pltpu.SemaphoreType.DMA,
            pltpu.SemaphoreType.DMA,
            pltpu.SemaphoreType.DMA,
        ],
    )(_emb_kernel)
    return k(step_idx, sig_idx, step_tab, sig_lines)


def kernel(step_levels, signal_levels, step_embedding, signal_embedding):
    step_idx = jnp.asarray(step_levels, dtype=jnp.int32)
    sig_idx = jnp.asarray(signal_levels, dtype=jnp.int32)
    sig_lines = (signal_embedding.T.reshape(8, 8, 8192, 128)
                 .transpose(0, 2, 1, 3).reshape(-1, 16))
    out_t = _lookup(step_idx, sig_idx, step_embedding, sig_lines)
    return out_t.T


# speedup vs baseline: 1.2503x; 1.0929x over previous
import functools

import jax
import jax.numpy as jnp
from jax import lax
from jax.experimental import pallas as pl
from jax.experimental.pallas import tpu as pltpu
from jax.experimental.pallas import tpu_sc as plsc

MODEL_DIM = 128
STEP_DIM = MODEL_DIM // 2
SIGNAL_DIM = MODEL_DIM - STEP_DIM
BATCH = 16384
STEP_VOCAB = 20

_NC, _NS = 2, 16
_NW = _NC * _NS
_BPW = BATCH // _NW

_LSPAN = 524232
_CCHUNK = 4
_NROUND = SIGNAL_DIM // _CCHUNK


def _emb_kernel(step_idx_hbm, sig_idx_hbm, step_tab_hbm, sig_lines_hbm,
                out_hbm, ridx_v, sidx_v, posl_v, lane_v, stab_v, lbuf_v,
                sobuf_v, robuf_v, sema, semb, osema, osemb, ssem):
    wid = lax.axis_index("s") * _NC + lax.axis_index("c")
    base = wid * _BPW
    iota16 = lax.iota(jnp.int32, 16)
    gsems = (sema, semb)
    osems = (osema, osemb)

    pltpu.sync_copy(sig_idx_hbm.at[pl.ds(base, _BPW)], ridx_v)
    pltpu.sync_copy(step_idx_hbm.at[pl.ds(base, _BPW)], sidx_v)
    pltpu.sync_copy(step_tab_hbm, stab_v)

    def _pos_body(k, carry):
        r = ridx_v[pl.ds(k * 16, 16)]
        posl_v[pl.ds(k * 16, 16)] = ((r >> 7) << 6) | ((r >> 4) & 7)
        lane_v[pl.ds(k * 16, 16)] = r & 15
        return carry

    lax.fori_loop(0, _BPW // 16, _pos_body, 0)

    def _fire(n):
        buf = n % 2
        cps = []
        for dc in range(_CCHUNK):
            c = n * _CCHUNK + dc
            src = sig_lines_hbm.at[
                pl.ds((c // 8) * 524288 + (c % 8) * 8, _LSPAN)]
            cps.append(pltpu.async_copy(
                src.at[posl_v],
                lbuf_v.at[buf, pl.ds(dc * _BPW, _BPW)], gsems[buf]))
        return cps

    pend = {0: _fire(0), 1: _fire(1)}

    def _step_body(k, carry):
        s = sidx_v[pl.ds(k * 16, 16)]
        for c in range(STEP_DIM):
            g = plsc.load_gather(stab_v, [s, jnp.full((16,), c, jnp.int32)])
            sobuf_v[c, pl.ds(k * 16, 16)] = g
        return carry

    lax.fori_loop(0, _BPW // 16, _step_body, 0)
    step_flush = pltpu.async_copy(
        sobuf_v, out_hbm.at[pl.ds(0, STEP_DIM), pl.ds(base, _BPW)], ssem)

    oflush = {}
    for n in range(_NROUND):
        buf = n % 2
        for cp in pend.pop(n):
            cp.wait()
        if n - 2 in oflush:
            oflush.pop(n - 2).wait()

        def _ext_body(k, carry, buf=buf):
            lane = lane_v[pl.ds(k * 16, 16)]
            row0 = k * 16 + iota16
            for dc in range(_CCHUNK):
                g = plsc.load_gather(
                    lbuf_v.at[buf], [row0 + dc * _BPW, lane])
                robuf_v[buf, dc, pl.ds(k * 16, 16)] = g
            return carry

        lax.fori_loop(0, _BPW // 16, _ext_body, 0)
        if n + 2 < _NROUND:
            pend[n + 2] = _fire(n + 2)
        oflush[n] = pltpu.async_copy(
            robuf_v.at[buf],
            out_hbm.at[pl.ds(STEP_DIM + n * _CCHUNK, _CCHUNK),
                       pl.ds(base, _BPW)], osems[buf])

    for n in sorted(oflush):
        oflush[n].wait()
    step_flush.wait()


@jax.jit
def _lookup(step_idx, sig_idx, step_tab, sig_lines):
    k = functools.partial(
        pl.kernel,
        out_type=jax.ShapeDtypeStruct((MODEL_DIM, BATCH), jnp.float32),
        mesh=plsc.VectorSubcoreMesh(core_axis_name="c", subcore_axis_name="s"),
        compiler_params=pltpu.CompilerParams(
            use_tc_tiling_on_sc=False, needs_layout_passes=False),
        scratch_types=[
            pltpu.VMEM((_BPW,), jnp.int32),
            pltpu.VMEM((_BPW,), jnp.int32),
            pltpu.VMEM((_BPW,), jnp.int32),
            pltpu.VMEM((_BPW,), jnp.int32),
            pltpu.VMEM((STEP_VOCAB, STEP_DIM), jnp.float32),
            pltpu.VMEM((2, _CCHUNK * _BPW, 16), jnp.float32),
            pltpu.VMEM((STEP_DIM, _BPW), jnp.float32),
            pltpu.VMEM((2, _CCHUNK, _BPW), jnp.float32),
            pltpu.SemaphoreType.DMA,
            pltpu.SemaphoreType.DMA,
            pltpu.SemaphoreType.DMA,
            pltpu.SemaphoreType.DMA,
            pltpu.SemaphoreType.DMA,
        ],
    )(_emb_kernel)
    return k(step_idx, sig_idx, step_tab, sig_lines)


def kernel(step_levels, signal_levels, step_embedding, signal_embedding):
    step_idx = jnp.asarray(step_levels, dtype=jnp.int32)
    sig_idx = jnp.asarray(signal_levels, dtype=jnp.int32)
    sig_lines = (signal_embedding.T.reshape(8, 8, 8192, 128)
                 .transpose(0, 2, 1, 3).reshape(-1, 16))
    out_t = _lookup(step_idx, sig_idx, step_embedding, sig_lines)
    return out_t.T.astype(jnp.bfloat16)


# speedup vs baseline: 1.2569x; 1.0053x over previous
import functools

import jax
import jax.numpy as jnp
from jax import lax
from jax.experimental import pallas as pl
from jax.experimental.pallas import tpu as pltpu
from jax.experimental.pallas import tpu_sc as plsc

MODEL_DIM = 128
STEP_DIM = MODEL_DIM // 2
SIGNAL_DIM = MODEL_DIM - STEP_DIM
BATCH = 16384
STEP_VOCAB = 20

_NC, _NS = 2, 16
_NW = _NC * _NS
_BPW = BATCH // _NW

_LSPAN = 524232
_CCHUNK = 2
_PDEPTH = 4
_NROUND = SIGNAL_DIM // _CCHUNK


def _emb_kernel(step_idx_hbm, sig_idx_hbm, step_tab_hbm, sig_lines_hbm,
                out_hbm, ridx_v, sidx_v, posl_v, lane_v, stab_v, lbuf_v,
                sobuf_v, robuf_v, sema, semb, semc, semd, osema, osemb, osemc, osemd, ssem):
    wid = lax.axis_index("s") * _NC + lax.axis_index("c")
    base = wid * _BPW
    iota16 = lax.iota(jnp.int32, 16)
    gsems = (sema, semb, semc, semd)
    osems = (osema, osemb, osemc, osemd)

    pltpu.sync_copy(sig_idx_hbm.at[pl.ds(base, _BPW)], ridx_v)
    pltpu.sync_copy(step_idx_hbm.at[pl.ds(base, _BPW)], sidx_v)
    pltpu.sync_copy(step_tab_hbm, stab_v)

    def _pos_body(k, carry):
        r = ridx_v[pl.ds(k * 16, 16)]
        posl_v[pl.ds(k * 16, 16)] = ((r >> 7) << 6) | ((r >> 4) & 7)
        lane_v[pl.ds(k * 16, 16)] = r & 15
        return carry

    lax.fori_loop(0, _BPW // 16, _pos_body, 0)

    def _fire(n):
        buf = n % _PDEPTH
        cps = []
        for dc in range(_CCHUNK):
            c = n * _CCHUNK + dc
            src = sig_lines_hbm.at[
                pl.ds((c // 8) * 524288 + (c % 8) * 8, _LSPAN)]
            cps.append(pltpu.async_copy(
                src.at[posl_v],
                lbuf_v.at[buf, pl.ds(dc * _BPW, _BPW)], gsems[buf]))
        return cps

    pend = {n: _fire(n) for n in range(_PDEPTH)}

    def _step_body(k, carry):
        s = sidx_v[pl.ds(k * 16, 16)]
        for c in range(STEP_DIM):
            g = plsc.load_gather(stab_v, [s, jnp.full((16,), c, jnp.int32)])
            sobuf_v[c, pl.ds(k * 16, 16)] = g
        return carry

    lax.fori_loop(0, _BPW // 16, _step_body, 0)
    step_flush = pltpu.async_copy(
        sobuf_v, out_hbm.at[pl.ds(0, STEP_DIM), pl.ds(base, _BPW)], ssem)

    oflush = {}
    for n in range(_NROUND):
        buf = n % _PDEPTH
        for cp in pend.pop(n):
            cp.wait()
        if n - _PDEPTH in oflush:
            oflush.pop(n - _PDEPTH).wait()

        def _ext_body(k, carry, buf=buf):
            lane = lane_v[pl.ds(k * 16, 16)]
            row0 = k * 16 + iota16
            for dc in range(_CCHUNK):
                g = plsc.load_gather(
                    lbuf_v.at[buf], [row0 + dc * _BPW, lane])
                robuf_v[buf, dc, pl.ds(k * 16, 16)] = g
            return carry

        lax.fori_loop(0, _BPW // 16, _ext_body, 0)
        if n + _PDEPTH < _NROUND:
            pend[n + _PDEPTH] = _fire(n + _PDEPTH)
        oflush[n] = pltpu.async_copy(
            robuf_v.at[buf],
            out_hbm.at[pl.ds(STEP_DIM + n * _CCHUNK, _CCHUNK),
                       pl.ds(base, _BPW)], osems[buf])

    for n in sorted(oflush):
        oflush[n].wait()
    step_flush.wait()


@jax.jit
def _lookup(step_idx, sig_idx, step_tab, sig_lines):
    k = functools.partial(
        pl.kernel,
        out_type=jax.ShapeDtypeStruct((MODEL_DIM, BATCH), jnp.float32),
        mesh=plsc.VectorSubcoreMesh(core_axis_name="c", subcore_axis_name="s"),
        compiler_params=pltpu.CompilerParams(
            use_tc_tiling_on_sc=False, needs_layout_passes=False),
        scratch_types=[
            pltpu.VMEM((_BPW,), jnp.int32),
            pltpu.VMEM((_BPW,), jnp.int32),
            pltpu.VMEM((_BPW,), jnp.int32),
            pltpu.VMEM((_BPW,), jnp.int32),
            pltpu.VMEM((STEP_VOCAB, STEP_DIM), jnp.float32),
            pltpu.VMEM((_PDEPTH, _CCHUNK * _BPW, 16), jnp.float32),
            pltpu.VMEM((STEP_DIM, _BPW), jnp.float32),
            pltpu.VMEM((_PDEPTH, _CCHUNK, _BPW), jnp.float32),
            pltpu.SemaphoreType.DMA,
            pltpu.SemaphoreType.DMA,
            pltpu.SemaphoreType.DMA,
            pltpu.SemaphoreType.DMA,
            pltpu.SemaphoreType.DMA,
            pltpu.SemaphoreType.DMA,
            pltpu.SemaphoreType.DMA,
            pltpu.SemaphoreType.DMA,
            pltpu.SemaphoreType.DMA,
        ],
    )(_emb_kernel)
    return k(step_idx, sig_idx, step_tab, sig_lines)


def kernel(step_levels, signal_levels, step_embedding, signal_embedding):
    step_idx = jnp.asarray(step_levels, dtype=jnp.int32)
    sig_idx = jnp.asarray(signal_levels, dtype=jnp.int32)
    sig_lines = (signal_embedding.T.reshape(8, 8, 8192, 128)
                 .transpose(0, 2, 1, 3).reshape(-1, 16))
    out_t = _lookup(step_idx, sig_idx, step_embedding, sig_lines)
    return out_t.T.astype(jnp.bfloat16)
